# baseline (speedup 1.0000x reference)
import jax
import jax.numpy as jnp
import numpy as np
from jax import lax
from jax.experimental import pallas as pl
from jax.experimental.pallas import tpu as pltpu
from jax.experimental.pallas import tpu_sc as plsc

_NUM_WORDS = 100000
_NUM_BUCKETS = 1 << 20
_MASK = _NUM_BUCKETS - 1
_K = 3
_D = 64
_B = 16384
_C = 5

_rs = np.random.RandomState(1139)
_HA = tuple(int(x) for x in _rs.randint(1, 21000, size=(_K,)))
_HB = tuple(int(x) for x in _rs.randint(0, _NUM_BUCKETS, size=(_K,)))

_NC, _NS, _L = 2, 16, 16
_NWK = _NC * _NS
_BPW = _B // _NWK
_CB = 64
_NCH = _BPW // _CB
_R = _C * _K


def _hash(ids, k):
    return (((ids * _HA[k]) & _MASK) + _HB[k]) & _MASK


def _body(tgt_hbm, ctx_hbm, tabT_hbm, impT_hbm, tabC_hbm, impC_hbm, out_hbm,
          tid_v, cid_v, idxT, widxT, idxC, widxC, wT, wC, rowsT, rowsC,
          out_v, sem):
    wid = lax.axis_index("s") * _NC + lax.axis_index("c")

    def chunk(ch, carry):
        base = wid * _BPW + ch * _CB
        pltpu.sync_copy(tgt_hbm.at[pl.ds(base, _CB)], tid_v)
        pltpu.sync_copy(ctx_hbm.at[pl.ds(base * _C, _CB * _C)], cid_v)
        for i in range(_CB // _L):
            lanes = lax.iota(jnp.int32, _L) + i * _L
            ids = tid_v[pl.ds(i * _L, _L)]
            for k in range(_K):
                idxT[k, pl.ds(i * _L, _L)] = _hash(ids, k)
                widxT[k, pl.ds(i * _L, _L)] = ids * _K + k
            for c in range(_C):
                cids = plsc.load_gather(cid_v, [lanes * _C + c])
                for k in range(_K):
                    r = c * _K + k
                    idxC[r, pl.ds(i * _L, _L)] = _hash(cids, k)
                    widxC[r, pl.ds(i * _L, _L)] = cids * _K + k
        cps = []
        for k in range(_K):
            cps.append(pltpu.async_copy(tabT_hbm.at[idxT.at[k]], rowsT.at[k], sem))
            cps.append(pltpu.async_copy(impT_hbm.at[widxT.at[k]], wT.at[k], sem))
        for r in range(_R):
            cps.append(pltpu.async_copy(tabC_hbm.at[idxC.at[r]], rowsC.at[r], sem))
            cps.append(pltpu.async_copy(impC_hbm.at[widxC.at[r]], wC.at[r], sem))
        for cp in cps:
            cp.wait()
        for g in range(_CB // _L):
            b0 = g * _L
            blane = lax.iota(jnp.int32, _L) + b0
            wTk = [wT[k, pl.ds(b0, _L)] for k in range(_K)]
            wCr = [wC[r, pl.ds(b0, _L)] for r in range(_R)]
            ksp = [jnp.full((_L,), k, jnp.int32) for k in range(_K)]
            rsp = [jnp.full((_L,), r, jnp.int32) for r in range(_R)]

            def dbody(d, accs):
                dsp = jnp.full((_L,), d, jnp.int32)
                we = wTk[0] * plsc.load_gather(rowsT, [ksp[0], blane, dsp])
                for k in range(1, _K):
                    we = we + wTk[k] * plsc.load_gather(rowsT, [ksp[k], blane, dsp])
                out = []
                for c in range(_C):
                    r0 = c * _K
                    ce = wCr[r0] * plsc.load_gather(rowsC, [rsp[r0], blane, dsp])
                    for k in range(1, _K):
                        r = r0 + k
                        ce = ce + wCr[r] * plsc.load_gather(rowsC, [rsp[r], blane, dsp])
                    out.append(accs[c] + we * ce)
                return tuple(out)

            accs = lax.fori_loop(
                0, _D, dbody,
                tuple(jnp.zeros((_L,), jnp.float32) for _ in range(_C)))
            for c in range(_C):
                plsc.store_scatter(out_v, [blane * _C + c], accs[c])
        pltpu.sync_copy(out_v, out_hbm.at[pl.ds(base * _C, _CB * _C)])
        return carry

    lax.fori_loop(0, _NCH, chunk, 0)


def kernel(target, context, table_target, imp_target, table_context, imp_context):
    tgt = target.reshape(_B).astype(jnp.int32)
    ctx = context.reshape(_B * _C).astype(jnp.int32)
    impT = imp_target.reshape(_NUM_WORDS * _K)
    impC = imp_context.reshape(_NUM_WORDS * _K)
    mesh = plsc.VectorSubcoreMesh(core_axis_name="c", subcore_axis_name="s",
                                  num_cores=_NC, num_subcores=_NS)
    out = pl.kernel(
        _body,
        out_type=jax.ShapeDtypeStruct((_B * _C,), jnp.float32),
        mesh=mesh,
        compiler_params=pltpu.CompilerParams(needs_layout_passes=False,
                                             use_tc_tiling_on_sc=False),
        scratch_types=[
            pltpu.VMEM((_CB,), jnp.int32),
            pltpu.VMEM((_CB * _C,), jnp.int32),
            pltpu.VMEM((_K, _CB), jnp.int32),
            pltpu.VMEM((_K, _CB), jnp.int32),
            pltpu.VMEM((_R, _CB), jnp.int32),
            pltpu.VMEM((_R, _CB), jnp.int32),
            pltpu.VMEM((_K, _CB), jnp.float32),
            pltpu.VMEM((_R, _CB), jnp.float32),
            pltpu.VMEM((_K, _CB, _D), jnp.float32),
            pltpu.VMEM((_R, _CB, _D), jnp.float32),
            pltpu.VMEM((_CB * _C,), jnp.float32),
            pltpu.SemaphoreType.DMA,
        ],
    )(tgt, ctx, table_target, impT, table_context, impC)
    return out.reshape(_B, _C)
